# baseline (speedup 1.0000x reference)
import functools

import jax
import jax.numpy as jnp
from jax.experimental import pallas as pl
from jax.experimental.pallas import tpu as pltpu

D_MODEL = 2048
N_FEATURE = 4096
D_SPACE = 64

TILE_M = 1024
M_TOTAL = 16384


def _tile_body(x_ref, w_ref, b_ref, emb_ref, out_ref):
    emb = emb_ref[...]
    sq = jnp.sum(emb * emb, axis=-1, keepdims=True)
    emb_norm = emb / jnp.maximum(jnp.sqrt(sq), 1e-12)

    h = jax.lax.dot_general(
        x_ref[...], w_ref[...],
        (((1,), (1,)), ((), ())),
        preferred_element_type=jnp.float32,
    ) + b_ref[...]
    out_ref[...] = jax.lax.dot_general(
        h, emb_norm,
        (((1,), (1,)), ((), ())),
        preferred_element_type=jnp.float32,
    )


def _router_body(x_hbm, w_hbm, b_hbm, emb_hbm, out_hbm):
    pipeline = pltpu.emit_pipeline(
        _tile_body,
        grid=(M_TOTAL // TILE_M,),
        in_specs=[
            pl.BlockSpec((TILE_M, D_MODEL), lambda m: (m, 0)),
            pl.BlockSpec((D_SPACE, D_MODEL), lambda m: (0, 0)),
            pl.BlockSpec((1, D_SPACE), lambda m: (0, 0)),
            pl.BlockSpec((N_FEATURE, D_SPACE), lambda m: (0, 0)),
        ],
        out_specs=[pl.BlockSpec((TILE_M, N_FEATURE), lambda m: (m, 0))],
        core_axis_name="core",
        dimension_semantics=(pltpu.PARALLEL,),
    )
    pipeline(x_hbm, w_hbm, b_hbm, emb_hbm, out_hbm)


@jax.jit
def kernel(x, W_proj, b_proj, neuron_emb):
    B, S, _ = x.shape
    M = B * S
    x2 = x.reshape(M, D_MODEL)
    emb = neuron_emb[:N_FEATURE]
    b2 = b_proj.reshape(1, D_SPACE)

    mesh = pltpu.create_tensorcore_mesh("core", num_cores=2)
    out = pl.kernel(
        _router_body,
        out_type=jax.ShapeDtypeStruct((M, N_FEATURE), jnp.float32),
        mesh=mesh,
    )(x2, W_proj, b2, emb)
    return out.reshape(B, S, N_FEATURE)
